# baseline (speedup 1.0000x reference)
import functools

import jax
import jax.numpy as jnp
from jax import lax
from jax.experimental import pallas as pl
from jax.experimental.pallas import tpu as pltpu
from jax.experimental.pallas import tpu_sc as plsc

VOCAB = 1000000
D = 64
B = 4096
L = 200
PAD = 0
N = B * L

NC = 2
NS = 16
NW = NC * NS
PER_W = N // NW
CHUNK = 128
NCHUNK = PER_W // CHUNK


def _gather_body(tok_hbm, table_hbm, out_hbm, idx_v, rows_v, sem):
    wid = lax.axis_index("s") * NC + lax.axis_index("c")
    base = wid * PER_W

    def body(j, carry):
        off = base + j * CHUNK
        pltpu.sync_copy(tok_hbm.at[pl.ds(off, CHUNK)], idx_v)
        pltpu.async_copy(table_hbm.at[idx_v], rows_v, sem).wait()
        pltpu.sync_copy(rows_v, out_hbm.at[pl.ds(off, CHUNK)])
        return carry

    lax.fori_loop(0, NCHUNK, body, 0)


@functools.partial(jax.jit, static_argnames=())
def _sc_gather(tokens_flat, table):
    mesh = plsc.VectorSubcoreMesh(core_axis_name="c", subcore_axis_name="s")
    k = functools.partial(
        pl.kernel,
        mesh=mesh,
        out_type=jax.ShapeDtypeStruct((N, D), jnp.float32),
        scratch_types=[
            pltpu.VMEM((CHUNK,), jnp.int32),
            pltpu.VMEM((CHUNK, D), jnp.float32),
            pltpu.SemaphoreType.DMA,
        ],
        compiler_params=pltpu.CompilerParams(use_tc_tiling_on_sc=False),
    )(_gather_body)
    return k(tokens_flat, table)


def _mask_body(tok_ref, pad_ref, seq_ref):
    pad_ref[...] = (tok_ref[...] == PAD).astype(jnp.int8)
    r = lax.broadcasted_iota(jnp.int32, (L, L), 0)
    c = lax.broadcasted_iota(jnp.int32, (L, L), 1)
    seq_ref[...] = (c > r).astype(jnp.int8)


def _tc_masks(tokens):
    return pl.pallas_call(
        _mask_body,
        out_shape=(
            jax.ShapeDtypeStruct((B, L), jnp.int8),
            jax.ShapeDtypeStruct((L, L), jnp.int8),
        ),
    )(tokens)


def kernel(tokens, table):
    features = _sc_gather(tokens.reshape(-1), table).reshape(B, L, D)
    pad8, seq8 = _tc_masks(tokens)
    padding_masks = pad8.astype(bool)[:, None, None, :]
    sequential_masks = seq8.astype(bool)
    return (features, padding_masks, sequential_masks)


# speedup vs baseline: 1.1980x; 1.1980x over previous
import functools

import jax
import jax.numpy as jnp
from jax import lax
from jax.experimental import pallas as pl
from jax.experimental.pallas import tpu as pltpu
from jax.experimental.pallas import tpu_sc as plsc

VOCAB = 1000000
D = 64
B = 4096
L = 200
PAD = 0
N = B * L

NC = 2
NS = 16
NW = NC * NS
PER_W = N // NW
CHUNK = 128
NCHUNK = PER_W // CHUNK
NBUF = 4
GROUPS = NCHUNK // NBUF


def _gather_body(tok_hbm, table_hbm, out_hbm, idx_v, bufs, sems):
    wid = lax.axis_index("s") * NC + lax.axis_index("c")
    base = wid * PER_W

    pltpu.sync_copy(tok_hbm.at[wid], idx_v)

    def start_gather(j, b):
        pltpu.async_copy(table_hbm.at[idx_v.at[j]], bufs[b], sems[b])

    def wait_gather(b):
        pltpu.make_async_copy(table_hbm.at[idx_v.at[0]], bufs[b], sems[b]).wait()

    for b in range(NBUF):
        start_gather(b, b)

    def body(g, carry):
        j0 = g * NBUF
        for b in range(NBUF):
            j = j0 + b
            wait_gather(b)
            pltpu.sync_copy(bufs[b], out_hbm.at[pl.ds(base + j * CHUNK, CHUNK)])

            @pl.when(g < GROUPS - 1)
            def _():
                start_gather(j + NBUF, b)

        return carry

    lax.fori_loop(0, GROUPS, body, 0)


def _sc_gather(tokens_blocked, table):
    mesh = plsc.VectorSubcoreMesh(core_axis_name="c", subcore_axis_name="s")
    k = functools.partial(
        pl.kernel,
        mesh=mesh,
        out_type=jax.ShapeDtypeStruct((N, D), jnp.float32),
        scratch_types=[
            pltpu.VMEM((NCHUNK, CHUNK), jnp.int32),
            [pltpu.VMEM((CHUNK, D), jnp.float32) for _ in range(NBUF)],
            [pltpu.SemaphoreType.DMA for _ in range(NBUF)],
        ],
        compiler_params=pltpu.CompilerParams(use_tc_tiling_on_sc=False),
    )(_gather_body)
    return k(tokens_blocked, table)


def _mask_body(tok_ref, pad_ref, seq_ref):
    pad_ref[...] = (tok_ref[...] == PAD).astype(jnp.int8)
    r = lax.broadcasted_iota(jnp.int32, (L, L), 0)
    c = lax.broadcasted_iota(jnp.int32, (L, L), 1)
    seq_ref[...] = (c > r).astype(jnp.int8)


def _tc_masks(tokens):
    return pl.pallas_call(
        _mask_body,
        out_shape=(
            jax.ShapeDtypeStruct((B, L), jnp.int8),
            jax.ShapeDtypeStruct((L, L), jnp.int8),
        ),
    )(tokens)


def kernel(tokens, table):
    tokens_blocked = tokens.reshape(NW, NCHUNK, CHUNK)
    features = _sc_gather(tokens_blocked, table).reshape(B, L, D)
    pad8, seq8 = _tc_masks(tokens)
    padding_masks = pad8.astype(bool)[:, None, None, :]
    sequential_masks = seq8.astype(bool)
    return (features, padding_masks, sequential_masks)
